# baseline (speedup 1.0000x reference)
import functools

import jax
import jax.numpy as jnp
from jax import lax
from jax.experimental import pallas as pl
from jax.experimental.pallas import tpu as pltpu
from jax.experimental.pallas import tpu_sc as plsc

NC = 2
NS = 16
NW = NC * NS
CHUNK = 112
LANES = 16
NBUF = 3
IBUF = 6


def _bcast_lane(g, l):
  idx = jnp.full((LANES, 1), l, jnp.int32)
  return lax.gather(
      g, idx,
      dimension_numbers=lax.GatherDimensionNumbers(
          offset_dims=(), collapsed_slice_dims=(0,), start_index_map=(0,)),
      slice_sizes=(1,), mode=lax.GatherScatterMode.PROMISE_IN_BOUNDS)


def _sc_segment_sum(x, src_f, dst_f, norm_f, zeros, n_pad, d, n0, n1):
  rows_per_tile = n_pad // NS
  epw0 = n0 * CHUNK
  epw1 = n1 * CHUNK

  mesh = plsc.VectorSubcoreMesh(core_axis_name="c", subcore_axis_name="s")

  @functools.partial(
      pl.kernel,
      out_type=jax.ShapeDtypeStruct((NC, n_pad, d), jnp.float32),
      mesh=mesh,
      scratch_types=[
          [pltpu.VMEM((CHUNK,), jnp.int32)] * IBUF,
          [pltpu.VMEM((CHUNK,), jnp.int32)] * IBUF,
          [pltpu.VMEM((CHUNK,), jnp.float32)] * IBUF,
          [pltpu.VMEM((CHUNK, d), jnp.float32)] * NBUF,
          pltpu.VMEM_SHARED((n_pad, d), jnp.float32),
          [pltpu.SemaphoreType.DMA] * IBUF,
          [pltpu.SemaphoreType.DMA] * NBUF,
          [pltpu.SemaphoreType.DMA] * NBUF,
      ],
  )
  def seg_kernel(x_hbm, src_hbm, dst_hbm, norm_hbm, z_hbm, out_hbm,
                 src_r, dst_r, norm_r, rows_r, agg_sh, isem, gsem, ssem):
    c = lax.axis_index("c")
    s = lax.axis_index("s")
    wid = c * NS + s
    base = jnp.where(c == 0, s * epw0, NS * epw0 + s * epw1)
    myn = jnp.where(c == 0, n0, n1)
    del wid

    def idx_copies(j, q):
      sl = pl.ds(base + j * CHUNK, CHUNK)
      return (pltpu.make_async_copy(src_hbm.at[sl], src_r[q], isem[q]),
              pltpu.make_async_copy(dst_hbm.at[sl], dst_r[q], isem[q]),
              pltpu.make_async_copy(norm_hbm.at[sl], norm_r[q], isem[q]))

    def idx_start(j, q):
      for cp in idx_copies(j, q):
        cp.start()

    def idx_wait(j, q):
      for cp in idx_copies(j, q):
        cp.wait()

    def gather_start(j, q, b):
      del j
      pltpu.async_copy(x_hbm.at[src_r[q]], rows_r[b], gsem[b])

    def gather_wait(j, q, b):
      del j
      pltpu.make_async_copy(x_hbm.at[src_r[q]], rows_r[b], gsem[b]).wait()

    def scatter_start(j, q, b):
      del j
      pltpu.async_copy(rows_r[b], agg_sh.at[dst_r[q]], ssem[b], add=True)

    def scatter_wait(j, q, b):
      del j
      pltpu.make_async_copy(rows_r[b], agg_sh.at[dst_r[q]], ssem[b]).wait()

    idx_start(0, 0)
    idx_start(1, 1)
    idx_start(2, 2)
    idx_start(3, 3)
    idx_wait(0, 0)
    gather_start(0, 0, 0)
    idx_wait(1, 1)
    gather_start(1, 1, 1)

    row0 = s * rows_per_tile
    pltpu.sync_copy(z_hbm.at[pl.ds(row0, rows_per_tile)],
                    agg_sh.at[pl.ds(row0, rows_per_tile)])
    plsc.subcore_barrier()

    def slot(j, qs):
      bs = qs % NBUF
      b2 = (bs + 2) % NBUF

      gather_wait(j, qs, bs)

      def group_body(gi, carry):
        g = norm_r[qs][pl.ds(gi * LANES, LANES)]
        for l in range(LANES):
          nb = _bcast_lane(g, l)
          r = gi * LANES + l
          for k in range(d // LANES):
            sl = pl.ds(k * LANES, LANES)
            rows_r[bs][r, sl] = rows_r[bs][r, sl] * nb
        return carry

      lax.fori_loop(0, CHUNK // LANES, group_body, 0)

      scatter_start(j, qs, bs)

      @pl.when(j >= 1)
      def _():
        scatter_wait(j - 1, (qs - 1) % IBUF, b2)

      @pl.when(j + 2 < myn)
      def _():
        idx_wait(j + 2, (qs + 2) % IBUF)
        gather_start(j + 2, (qs + 2) % IBUF, b2)

      @pl.when(j + 4 < myn)
      def _():
        idx_start(j + 4, (qs + 4) % IBUF)

    def outer(t, carry):
      for qs in range(IBUF):
        slot(t * IBUF + qs, qs)
      return carry

    lax.fori_loop(0, jnp.where(c == 0, n0 // IBUF, n1 // IBUF), outer, 0)

    @pl.when(c == 0)
    def _():
      scatter_wait(n0 - 1, (n0 - 1) % IBUF, (n0 - 1) % NBUF)

    @pl.when(c == 1)
    def _():
      scatter_wait(n1 - 1, (n1 - 1) % IBUF, (n1 - 1) % NBUF)
    plsc.subcore_barrier()

    pltpu.sync_copy(agg_sh.at[pl.ds(row0, rows_per_tile)],
                    out_hbm.at[c].at[pl.ds(row0, rows_per_tile)])

  return seg_kernel(x, src_f, dst_f, norm_f, zeros)


def _tc_dense(partials, W_rgc, W_u, W_i, n_nodes, num_users, d, out_dim):
  rows = 1000
  grid = n_nodes // rows
  user_blocks = num_users // rows

  def body(p_ref, w1_ref, wu_ref, wi_ref, out_ref):
    agg = p_ref[0] + p_ref[1]
    h = jnp.maximum(
        jnp.dot(agg, w1_ref[...], preferred_element_type=jnp.float32), 0.0)
    u = jnp.dot(h, wu_ref[...], preferred_element_type=jnp.float32)
    v = jnp.dot(h, wi_ref[...], preferred_element_type=jnp.float32)
    sel = pl.program_id(0) < user_blocks
    out_ref[...] = jnp.maximum(jnp.where(sel, u, v), 0.0)

  h = W_rgc.shape[1]
  return pl.pallas_call(
      body,
      grid=(grid,),
      in_specs=[
          pl.BlockSpec((2, rows, d), lambda i: (0, i, 0)),
          pl.BlockSpec((d, h), lambda i: (0, 0)),
          pl.BlockSpec((h, out_dim), lambda i: (0, 0)),
          pl.BlockSpec((h, out_dim), lambda i: (0, 0)),
      ],
      out_specs=pl.BlockSpec((rows, out_dim), lambda i: (i, 0)),
      out_shape=jax.ShapeDtypeStruct((n_nodes, out_dim), jnp.float32),
  )(partials, W_rgc, W_u, W_i)


def kernel(x, edge_index, edge_norm, W_rgc, W_u, W_i):
  n_nodes, d = x.shape
  e = edge_index.shape[1]
  num_users = 2000
  out_dim = W_u.shape[1]

  nchunk = -(-e // (NW * CHUNK))
  nchunk = -(-nchunk // IBUF) * IBUF
  n1 = 36
  n0 = 2 * nchunk - n1
  e_pad = NW * nchunk * CHUNK
  pad = e_pad - e

  src = edge_index[0]
  dst = edge_index[1]
  src_f = jnp.concatenate([src, jnp.zeros((pad,), jnp.int32)])
  dst_f = jnp.concatenate([dst, jnp.zeros((pad,), jnp.int32)])
  norm_f = jnp.concatenate([edge_norm, jnp.zeros((pad,), jnp.float32)])

  n_pad = -(-n_nodes // (8 * NS)) * (8 * NS)
  zeros = jnp.zeros((n_pad, d), jnp.float32)

  partials = _sc_segment_sum(x, src_f, dst_f, norm_f, zeros,
                             n_pad, d, n0, n1)
  out = _tc_dense(partials, W_rgc, W_u, W_i, n_nodes, num_users, d, out_dim)
  return (out[:num_users], out[num_users:])


# speedup vs baseline: 1.0064x; 1.0064x over previous
import functools

import jax
import jax.numpy as jnp
from jax import lax
from jax.experimental import pallas as pl
from jax.experimental.pallas import tpu as pltpu
from jax.experimental.pallas import tpu_sc as plsc

NC = 2
NS = 16
NW = NC * NS
CHUNK = 112
LANES = 16
NBUF = 3
IBUF = 6


def _bcast_lane(g, l):
  idx = jnp.full((LANES, 1), l, jnp.int32)
  return lax.gather(
      g, idx,
      dimension_numbers=lax.GatherDimensionNumbers(
          offset_dims=(), collapsed_slice_dims=(0,), start_index_map=(0,)),
      slice_sizes=(1,), mode=lax.GatherScatterMode.PROMISE_IN_BOUNDS)


def _sc_segment_sum(x, src_f, dst_f, norm_f, zeros, n_pad, d, n0, n1):
  rows_per_tile = n_pad // NS
  epw0 = n0 * CHUNK
  epw1 = n1 * CHUNK

  mesh = plsc.VectorSubcoreMesh(core_axis_name="c", subcore_axis_name="s")

  @functools.partial(
      pl.kernel,
      out_type=jax.ShapeDtypeStruct((NC, n_pad, d), jnp.float32),
      mesh=mesh,
      scratch_types=[
          [pltpu.VMEM((CHUNK,), jnp.int32)] * IBUF,
          [pltpu.VMEM((CHUNK,), jnp.int32)] * IBUF,
          [pltpu.VMEM((CHUNK,), jnp.float32)] * IBUF,
          [pltpu.VMEM((CHUNK, d), jnp.float32)] * NBUF,
          pltpu.VMEM_SHARED((n_pad, d), jnp.float32),
          [pltpu.SemaphoreType.DMA] * IBUF,
          [pltpu.SemaphoreType.DMA] * NBUF,
          [pltpu.SemaphoreType.DMA] * NBUF,
      ],
  )
  def seg_kernel(x_hbm, src_hbm, dst_hbm, norm_hbm, z_hbm, out_hbm,
                 src_r, dst_r, norm_r, rows_r, agg_sh, isem, gsem, ssem):
    c = lax.axis_index("c")
    s = lax.axis_index("s")
    wid = c * NS + s
    base = jnp.where(c == 0, s * epw0, NS * epw0 + s * epw1)
    myn = jnp.where(c == 0, n0, n1)
    del wid

    def idx_copies(j, q):
      sl = pl.ds(base + j * CHUNK, CHUNK)
      return (pltpu.make_async_copy(src_hbm.at[sl], src_r[q], isem[q]),
              pltpu.make_async_copy(dst_hbm.at[sl], dst_r[q], isem[q]),
              pltpu.make_async_copy(norm_hbm.at[sl], norm_r[q], isem[q]))

    def idx_start(j, q):
      for cp in idx_copies(j, q):
        cp.start()

    def idx_wait(j, q):
      for cp in idx_copies(j, q):
        cp.wait()

    def gather_start(j, q, b):
      del j
      pltpu.async_copy(x_hbm.at[src_r[q]], rows_r[b], gsem[b])

    def gather_wait(j, q, b):
      del j
      pltpu.make_async_copy(x_hbm.at[src_r[q]], rows_r[b], gsem[b]).wait()

    def scatter_start(j, q, b):
      del j
      pltpu.async_copy(rows_r[b], agg_sh.at[dst_r[q]], ssem[b], add=True)

    def scatter_wait(j, q, b):
      del j
      pltpu.make_async_copy(rows_r[b], agg_sh.at[dst_r[q]], ssem[b]).wait()

    idx_start(0, 0)
    idx_start(1, 1)
    idx_start(2, 2)
    idx_start(3, 3)
    idx_wait(0, 0)
    gather_start(0, 0, 0)
    idx_wait(1, 1)
    gather_start(1, 1, 1)

    row0 = s * rows_per_tile
    pltpu.sync_copy(z_hbm.at[pl.ds(row0, rows_per_tile)],
                    agg_sh.at[pl.ds(row0, rows_per_tile)])
    plsc.subcore_barrier()

    def slot(j, qs):
      bs = qs % NBUF
      b2 = (bs + 2) % NBUF

      gather_wait(j, qs, bs)

      def group_body(gi, carry):
        g = norm_r[qs][pl.ds(gi * LANES, LANES)]
        for l in range(LANES):
          nb = _bcast_lane(g, l)
          r = gi * LANES + l
          for k in range(d // LANES):
            sl = pl.ds(k * LANES, LANES)
            rows_r[bs][r, sl] = rows_r[bs][r, sl] * nb
        return carry

      lax.fori_loop(0, CHUNK // LANES, group_body, 0)

      scatter_start(j, qs, bs)

      @pl.when(j >= 1)
      def _():
        scatter_wait(j - 1, (qs - 1) % IBUF, b2)

      @pl.when(j + 2 < myn)
      def _():
        idx_wait(j + 2, (qs + 2) % IBUF)
        gather_start(j + 2, (qs + 2) % IBUF, b2)

      @pl.when(j + 4 < myn)
      def _():
        idx_start(j + 4, (qs + 4) % IBUF)

    def outer(t, carry):
      for qs in range(IBUF):
        slot(t * IBUF + qs, qs)
      return carry

    lax.fori_loop(0, jnp.where(c == 0, n0 // IBUF, n1 // IBUF), outer, 0)

    @pl.when(c == 0)
    def _():
      scatter_wait(n0 - 1, (n0 - 1) % IBUF, (n0 - 1) % NBUF)

    @pl.when(c == 1)
    def _():
      scatter_wait(n1 - 1, (n1 - 1) % IBUF, (n1 - 1) % NBUF)
    plsc.subcore_barrier()

    pltpu.sync_copy(agg_sh.at[pl.ds(row0, rows_per_tile)],
                    out_hbm.at[c].at[pl.ds(row0, rows_per_tile)])

  return seg_kernel(x, src_f, dst_f, norm_f, zeros)


def _tc_dense(partials, W_rgc, W_u, W_i, n_nodes, num_users, d, out_dim):
  rows = 1000
  grid = n_nodes // rows
  user_blocks = num_users // rows

  def body(p_ref, w1_ref, wu_ref, wi_ref, out_ref):
    agg = p_ref[0] + p_ref[1]
    h = jnp.maximum(
        jnp.dot(agg, w1_ref[...], preferred_element_type=jnp.float32), 0.0)
    u = jnp.dot(h, wu_ref[...], preferred_element_type=jnp.float32)
    v = jnp.dot(h, wi_ref[...], preferred_element_type=jnp.float32)
    sel = pl.program_id(0) < user_blocks
    out_ref[...] = jnp.maximum(jnp.where(sel, u, v), 0.0)

  h = W_rgc.shape[1]
  return pl.pallas_call(
      body,
      grid=(grid,),
      in_specs=[
          pl.BlockSpec((2, rows, d), lambda i: (0, i, 0)),
          pl.BlockSpec((d, h), lambda i: (0, 0)),
          pl.BlockSpec((h, out_dim), lambda i: (0, 0)),
          pl.BlockSpec((h, out_dim), lambda i: (0, 0)),
      ],
      out_specs=pl.BlockSpec((rows, out_dim), lambda i: (i, 0)),
      out_shape=jax.ShapeDtypeStruct((n_nodes, out_dim), jnp.float32),
  )(partials, W_rgc, W_u, W_i)


def kernel(x, edge_index, edge_norm, W_rgc, W_u, W_i):
  n_nodes, d = x.shape
  e = edge_index.shape[1]
  num_users = 2000
  out_dim = W_u.shape[1]

  nchunk = -(-e // (NW * CHUNK))
  nchunk = -(-nchunk // IBUF) * IBUF
  n1 = 42
  n0 = 2 * nchunk - n1
  e_pad = NW * nchunk * CHUNK
  pad = e_pad - e

  src = edge_index[0]
  dst = edge_index[1]
  src_f = jnp.concatenate([src, jnp.zeros((pad,), jnp.int32)])
  dst_f = jnp.concatenate([dst, jnp.zeros((pad,), jnp.int32)])
  norm_f = jnp.concatenate([edge_norm, jnp.zeros((pad,), jnp.float32)])

  n_pad = -(-n_nodes // (8 * NS)) * (8 * NS)
  zeros = jnp.zeros((n_pad, d), jnp.float32)

  partials = _sc_segment_sum(x, src_f, dst_f, norm_f, zeros,
                             n_pad, d, n0, n1)
  out = _tc_dense(partials, W_rgc, W_u, W_i, n_nodes, num_users, d, out_dim)
  return (out[:num_users], out[num_users:])
